# baseline (speedup 1.0000x reference)
import functools

import jax
import jax.numpy as jnp
from jax import lax
from jax.experimental import pallas as pl
from jax.experimental.pallas import tpu as pltpu
from jax.experimental.pallas import tpu_sc as plsc

N_CORES = 2
N_SUBCORES = 16
N_TILES = N_CORES * N_SUBCORES
CHUNK = 128
SLOW_CORE = 0
FRAC_SLOW = 0.51


def _rows_acc(n_nodes):
    step = N_SUBCORES * 8
    return ((n_nodes + 1 + step - 1) // step) * step


def _aggregate_body(nc0, nc1, rows_per_tile, feat, srcp, dstp, zeros_f,
                    out_sum, srcv, dstv, rows0, acc, gsem0):
    cid = lax.axis_index("c")
    sid = lax.axis_index("s")
    wid = cid * N_SUBCORES + sid

    base = sid * rows_per_tile
    pltpu.sync_copy(zeros_f, acc.at[pl.ds(base, rows_per_tile)])
    pltpu.sync_copy(srcp.at[wid], srcv)
    pltpu.sync_copy(dstp.at[wid], dstv)
    plsc.subcore_barrier()

    def chunk_step(j, carry):
        pltpu.async_copy(feat.at[srcv.at[j]], rows0, gsem0).wait()
        pltpu.sync_copy(rows0, acc.at[dstv.at[j]], add=True)
        return carry

    n_self = lax.select(cid == 0, nc0, nc1)
    lax.fori_loop(0, n_self, chunk_step, 0)
    plsc.subcore_barrier()

    pltpu.sync_copy(acc.at[pl.ds(base, rows_per_tile)],
                    out_sum.at[cid, pl.ds(base, rows_per_tile)])


def _make_aggregate(n_nodes, d, nc0, nc1):
    rows = _rows_acc(n_nodes)
    rows_per_tile = rows // N_SUBCORES
    nc_max = max(nc0, nc1)
    mesh = plsc.VectorSubcoreMesh(core_axis_name="c", subcore_axis_name="s")
    out_type = jax.ShapeDtypeStruct((N_CORES, rows, d), jnp.float32)
    scratch = [
        pltpu.VMEM((nc_max, CHUNK), jnp.int32),
        pltpu.VMEM((nc_max, CHUNK), jnp.int32),
        pltpu.VMEM((CHUNK, d), jnp.float32),
        pltpu.VMEM_SHARED((rows, d), jnp.float32),
        pltpu.SemaphoreType.DMA,
    ]
    body = functools.partial(_aggregate_body, nc0, nc1, rows_per_tile)
    return pl.kernel(body, out_type=out_type, mesh=mesh,
                     scratch_types=scratch)


def _count_body(n_chunks, rows_per_tile, d, dstp, zeros_f, ones_h, out_cnt,
                dstv, onesv, cntacc):
    cid = lax.axis_index("c")
    sid = lax.axis_index("s")
    wid = cid * N_SUBCORES + sid

    base = sid * rows_per_tile
    pltpu.sync_copy(zeros_f, cntacc.at[pl.ds(base, rows_per_tile)])
    pltpu.sync_copy(ones_h, onesv)
    pltpu.sync_copy(dstp.at[wid], dstv)
    plsc.subcore_barrier()

    def chunk_step(j, carry):
        pltpu.sync_copy(onesv, cntacc.at[dstv.at[j]], add=True)
        return carry

    lax.fori_loop(0, n_chunks, chunk_step, 0)
    plsc.subcore_barrier()

    pltpu.sync_copy(cntacc.at[pl.ds(base, rows_per_tile)],
                    out_cnt.at[cid, pl.ds(base, rows_per_tile)])


def _make_count(n_nodes, d, n_chunks):
    rows = _rows_acc(n_nodes)
    rows_per_tile = rows // N_SUBCORES
    mesh = plsc.VectorSubcoreMesh(core_axis_name="c", subcore_axis_name="s")
    out_type = jax.ShapeDtypeStruct((N_CORES, rows, d), jnp.float32)
    scratch = [
        pltpu.VMEM((n_chunks, CHUNK), jnp.int32),
        pltpu.VMEM((CHUNK, d), jnp.float32),
        pltpu.VMEM_SHARED((rows, d), jnp.float32),
    ]
    body = functools.partial(_count_body, n_chunks, rows_per_tile, d)
    return pl.kernel(body, out_type=out_type, mesh=mesh,
                     scratch_types=scratch)


def _dense_body(n_nodes, relu, p_ref, c_ref, x_ref, wl_ref, wr_ref, b_ref, o_ref):
    s = p_ref[0, :n_nodes, :] + p_ref[1, :n_nodes, :]
    cnt = c_ref[0, :n_nodes, 0] + c_ref[1, :n_nodes, 0]
    mean = s / jnp.maximum(cnt, 1.0)[:, None]
    dn = (((1,), (1,)), ((), ()))
    out = (lax.dot_general(mean, wl_ref[...], dn, preferred_element_type=jnp.float32)
           + lax.dot_general(x_ref[...], wr_ref[...], dn, preferred_element_type=jnp.float32)
           + b_ref[...])
    o_ref[...] = jnp.maximum(out, 0.0) if relu else out


def _dense(p, cnt, x, w_l, w_r, b, relu):
    n_nodes, d = x.shape
    return pl.pallas_call(
        functools.partial(_dense_body, n_nodes, relu),
        out_shape=jax.ShapeDtypeStruct((n_nodes, d), jnp.float32),
    )(p, cnt, x, w_l, w_r, b.reshape(1, -1))


def kernel(x, edge_index, W1_l, W1_r, b1, W2_l, W2_r, b2):
    n_nodes, d = x.shape
    e = edge_index.shape[1]
    src = edge_index[0].astype(jnp.int32)
    dst = edge_index[1].astype(jnp.int32)

    per_tile_cap = N_SUBCORES * CHUNK
    nc_slow = -(-int(e * FRAC_SLOW) // per_tile_cap)
    nc_fast = -(-(e - N_SUBCORES * nc_slow * CHUNK) // per_tile_cap)
    nc0, nc1 = (nc_slow, nc_fast) if SLOW_CORE == 0 else (nc_fast, nc_slow)
    nc_max = max(nc0, nc1)
    cap0 = N_SUBCORES * nc0 * CHUNK
    cap1 = N_SUBCORES * nc1 * CHUNK
    srcf = jnp.concatenate([src, jnp.zeros((cap0 + cap1 - e,), jnp.int32)])
    dstf = jnp.concatenate([dst, jnp.full((cap0 + cap1 - e,), n_nodes, jnp.int32)])

    def _layout(flat):
        a = flat[:cap0].reshape(N_SUBCORES, nc0, CHUNK)
        b = flat[cap0:].reshape(N_SUBCORES, nc1, CHUNK)
        a = jnp.pad(a, ((0, 0), (0, nc_max - nc0), (0, 0)))
        b = jnp.pad(b, ((0, 0), (0, nc_max - nc1), (0, 0)))
        return jnp.concatenate([a, b], axis=0)

    srcp = _layout(srcf)
    dstp = _layout(dstf)

    n_chunks_u = -(-e // (N_TILES * CHUNK))
    e_pad_u = N_TILES * n_chunks_u * CHUNK
    dstp_u = jnp.concatenate(
        [dst, jnp.full((e_pad_u - e,), n_nodes, jnp.int32)]
    ).reshape(N_TILES, n_chunks_u, CHUNK)

    agg = _make_aggregate(n_nodes, d, nc0, nc1)
    count = _make_count(n_nodes, d, n_chunks_u)
    rows_per_tile = _rows_acc(n_nodes) // N_SUBCORES
    zeros_f = jnp.zeros((rows_per_tile, d), jnp.float32)
    ones_h = jnp.ones((CHUNK, d), jnp.float32)

    cnt = count(dstp_u, zeros_f, ones_h)
    p1 = agg(x, srcp, dstp, zeros_f)
    h = _dense(p1, cnt, x, W1_l, W1_r, b1, relu=True)
    p2 = agg(h, srcp, dstp, zeros_f)
    return _dense(p2, cnt, h, W2_l, W2_r, b2, relu=False)


# speedup vs baseline: 1.0810x; 1.0810x over previous
import functools

import jax
import jax.numpy as jnp
from jax import lax
from jax.experimental import pallas as pl
from jax.experimental.pallas import tpu as pltpu
from jax.experimental.pallas import tpu_sc as plsc

N_CORES = 2
N_SUBCORES = 16
N_TILES = N_CORES * N_SUBCORES
CHUNK = 128
SLOW_CORE = 0
FRAC_SLOW = 0.56


def _rows_acc(n_nodes):
    step = N_SUBCORES * 8
    return ((n_nodes + 1 + step - 1) // step) * step


def _aggregate_body(nc0, nc1, rows_per_tile, feat, srcp, dstp, zeros_f,
                    out_sum, srcv, dstv, rows0, acc, gsem0):
    cid = lax.axis_index("c")
    sid = lax.axis_index("s")
    wid = cid * N_SUBCORES + sid

    base = sid * rows_per_tile
    pltpu.sync_copy(zeros_f, acc.at[pl.ds(base, rows_per_tile)])
    pltpu.sync_copy(srcp.at[wid], srcv)
    pltpu.sync_copy(dstp.at[wid], dstv)
    plsc.subcore_barrier()

    def chunk_step(j, carry):
        pltpu.async_copy(feat.at[srcv.at[j]], rows0, gsem0).wait()
        pltpu.sync_copy(rows0, acc.at[dstv.at[j]], add=True)
        return carry

    n_self = lax.select(cid == 0, nc0, nc1)
    lax.fori_loop(0, n_self, chunk_step, 0)
    plsc.subcore_barrier()

    pltpu.sync_copy(acc.at[pl.ds(base, rows_per_tile)],
                    out_sum.at[cid, pl.ds(base, rows_per_tile)])


def _make_aggregate(n_nodes, d, nc0, nc1):
    rows = _rows_acc(n_nodes)
    rows_per_tile = rows // N_SUBCORES
    nc_max = max(nc0, nc1)
    mesh = plsc.VectorSubcoreMesh(core_axis_name="c", subcore_axis_name="s")
    out_type = jax.ShapeDtypeStruct((N_CORES, rows, d), jnp.float32)
    scratch = [
        pltpu.VMEM((nc_max, CHUNK), jnp.int32),
        pltpu.VMEM((nc_max, CHUNK), jnp.int32),
        pltpu.VMEM((CHUNK, d), jnp.float32),
        pltpu.VMEM_SHARED((rows, d), jnp.float32),
        pltpu.SemaphoreType.DMA,
    ]
    body = functools.partial(_aggregate_body, nc0, nc1, rows_per_tile)
    return pl.kernel(body, out_type=out_type, mesh=mesh,
                     scratch_types=scratch)


def _count_body(n_chunks, rows_per_tile, d, dstp, zeros_f, ones_h, out_cnt,
                dstv, onesv, cntacc):
    cid = lax.axis_index("c")
    sid = lax.axis_index("s")
    wid = cid * N_SUBCORES + sid

    base = sid * rows_per_tile
    pltpu.sync_copy(zeros_f, cntacc.at[pl.ds(base, rows_per_tile)])
    pltpu.sync_copy(ones_h, onesv)
    pltpu.sync_copy(dstp.at[wid], dstv)
    plsc.subcore_barrier()

    def chunk_step(j, carry):
        pltpu.sync_copy(onesv, cntacc.at[dstv.at[j]], add=True)
        return carry

    lax.fori_loop(0, n_chunks, chunk_step, 0)
    plsc.subcore_barrier()

    pltpu.sync_copy(cntacc.at[pl.ds(base, rows_per_tile)],
                    out_cnt.at[cid, pl.ds(base, rows_per_tile)])


def _make_count(n_nodes, d, n_chunks):
    rows = _rows_acc(n_nodes)
    rows_per_tile = rows // N_SUBCORES
    mesh = plsc.VectorSubcoreMesh(core_axis_name="c", subcore_axis_name="s")
    out_type = jax.ShapeDtypeStruct((N_CORES, rows, d), jnp.float32)
    scratch = [
        pltpu.VMEM((n_chunks, CHUNK), jnp.int32),
        pltpu.VMEM((CHUNK, d), jnp.float32),
        pltpu.VMEM_SHARED((rows, d), jnp.float32),
    ]
    body = functools.partial(_count_body, n_chunks, rows_per_tile, d)
    return pl.kernel(body, out_type=out_type, mesh=mesh,
                     scratch_types=scratch)


def _dense_body(n_nodes, relu, p_ref, c_ref, x_ref, wl_ref, wr_ref, b_ref, o_ref):
    s = p_ref[0, :n_nodes, :] + p_ref[1, :n_nodes, :]
    cnt = c_ref[0, :n_nodes, 0] + c_ref[1, :n_nodes, 0]
    mean = s / jnp.maximum(cnt, 1.0)[:, None]
    dn = (((1,), (1,)), ((), ()))
    out = (lax.dot_general(mean, wl_ref[...], dn, preferred_element_type=jnp.float32)
           + lax.dot_general(x_ref[...], wr_ref[...], dn, preferred_element_type=jnp.float32)
           + b_ref[...])
    o_ref[...] = jnp.maximum(out, 0.0) if relu else out


def _dense(p, cnt, x, w_l, w_r, b, relu):
    n_nodes, d = x.shape
    return pl.pallas_call(
        functools.partial(_dense_body, n_nodes, relu),
        out_shape=jax.ShapeDtypeStruct((n_nodes, d), jnp.float32),
    )(p, cnt, x, w_l, w_r, b.reshape(1, -1))


def kernel(x, edge_index, W1_l, W1_r, b1, W2_l, W2_r, b2):
    n_nodes, d = x.shape
    e = edge_index.shape[1]
    src = edge_index[0].astype(jnp.int32)
    dst = edge_index[1].astype(jnp.int32)

    per_tile_cap = N_SUBCORES * CHUNK
    nc_slow = -(-int(e * FRAC_SLOW) // per_tile_cap)
    nc_fast = -(-(e - N_SUBCORES * nc_slow * CHUNK) // per_tile_cap)
    nc0, nc1 = (nc_slow, nc_fast) if SLOW_CORE == 0 else (nc_fast, nc_slow)
    nc_max = max(nc0, nc1)
    cap0 = N_SUBCORES * nc0 * CHUNK
    cap1 = N_SUBCORES * nc1 * CHUNK
    srcf = jnp.concatenate([src, jnp.zeros((cap0 + cap1 - e,), jnp.int32)])
    dstf = jnp.concatenate([dst, jnp.full((cap0 + cap1 - e,), n_nodes, jnp.int32)])

    def _layout(flat):
        a = flat[:cap0].reshape(N_SUBCORES, nc0, CHUNK)
        b = flat[cap0:].reshape(N_SUBCORES, nc1, CHUNK)
        a = jnp.pad(a, ((0, 0), (0, nc_max - nc0), (0, 0)))
        b = jnp.pad(b, ((0, 0), (0, nc_max - nc1), (0, 0)))
        return jnp.concatenate([a, b], axis=0)

    srcp = _layout(srcf)
    dstp = _layout(dstf)

    n_chunks_u = -(-e // (N_TILES * CHUNK))
    e_pad_u = N_TILES * n_chunks_u * CHUNK
    dstp_u = jnp.concatenate(
        [dst, jnp.full((e_pad_u - e,), n_nodes, jnp.int32)]
    ).reshape(N_TILES, n_chunks_u, CHUNK)

    agg = _make_aggregate(n_nodes, d, nc0, nc1)
    count = _make_count(n_nodes, d, n_chunks_u)
    rows_per_tile = _rows_acc(n_nodes) // N_SUBCORES
    zeros_f = jnp.zeros((rows_per_tile, d), jnp.float32)
    ones_h = jnp.ones((CHUNK, d), jnp.float32)

    cnt = count(dstp_u, zeros_f, ones_h)
    p1 = agg(x, srcp, dstp, zeros_f)
    h = _dense(p1, cnt, x, W1_l, W1_r, b1, relu=True)
    p2 = agg(h, srcp, dstp, zeros_f)
    return _dense(p2, cnt, h, W2_l, W2_r, b2, relu=False)
